# baseline (speedup 1.0000x reference)
import functools

import jax
import jax.numpy as jnp
from jax.experimental import pallas as pl
from jax.experimental.pallas import tpu as pltpu


def _hilo(s):
    hi = s.astype(jnp.bfloat16)
    lo = (s - hi.astype(jnp.float32)).astype(jnp.bfloat16)
    return jnp.concatenate([hi, lo], axis=1)


def _l1_body(g_ref, f_ref, w_ref, h1_ref, gb_ref, s_ref):
    @pl.when(pl.program_id(0) == 0)
    def _():
        s_ref[...] = jnp.dot(f_ref[...], w_ref[...],
                             preferred_element_type=jnp.float32)

    gblk = g_ref[...]
    m = jnp.dot(gblk, s_ref[...], preferred_element_type=jnp.float32)
    h1_ref[...] = jnp.tanh(m)
    gb_ref[...] = gblk.astype(jnp.bfloat16)


def _rest_body(gb_ref, h1_ref, w2_ref, w3_ref, h2_ref, h3_ref, adj_ref,
               s2_ref, s3_ref, h2s_ref, h3s_ref, *, p, block_m):
    i = pl.program_id(0)

    @pl.when(i == 0)
    def _():
        s2_ref[...] = _hilo(jnp.dot(h1_ref[...], w2_ref[...],
                                    preferred_element_type=jnp.float32))

    @pl.when(i < p)
    def _():
        acc = jnp.dot(gb_ref[...], s2_ref[...],
                      preferred_element_type=jnp.float32)
        k = h2_ref.shape[1]
        m2 = acc[:, :k] + acc[:, k:]
        h2_ref[...] = m2
        h2s_ref[pl.ds(i * block_m, block_m), :] = m2

    @pl.when(i == p)
    def _():
        s3_ref[...] = _hilo(jnp.dot(h2s_ref[...], w3_ref[...],
                                    preferred_element_type=jnp.float32))

    @pl.when(jnp.logical_and(i >= p, i < 2 * p))
    def _():
        acc = jnp.dot(gb_ref[...], s3_ref[...],
                      preferred_element_type=jnp.float32)
        k = h3_ref.shape[1]
        m3 = acc[:, :k] + acc[:, k:]
        h3_ref[...] = m3
        h3s_ref[pl.ds((i - p) * block_m, block_m), :] = m3

    @pl.when(i >= 2 * p)
    def _():
        a = h3s_ref[pl.ds((i - 2 * p) * block_m, block_m), :]
        adj_ref[...] = jax.lax.dot_general(
            a, h3s_ref[...], (((1,), (1,)), ((), ())),
            preferred_element_type=jnp.float32)


def kernel(g, f, W1, W2, W3):
    n = g.shape[0]
    d0 = f.shape[1]
    d1, d2, d3 = W1.shape[1], W2.shape[1], W3.shape[1]

    block_m = 400
    p = n // block_m

    h1, gb = pl.pallas_call(
        _l1_body,
        grid=(p,),
        in_specs=[
            pl.BlockSpec((block_m, n), lambda i: (i, 0)),
            pl.BlockSpec((n, d0), lambda i: (0, 0)),
            pl.BlockSpec((d0, d1), lambda i: (0, 0)),
        ],
        out_specs=[
            pl.BlockSpec((block_m, d1), lambda i: (i, 0)),
            pl.BlockSpec((block_m, n), lambda i: (i, 0)),
        ],
        out_shape=[
            jax.ShapeDtypeStruct((n, d1), jnp.float32),
            jax.ShapeDtypeStruct((n, n), jnp.bfloat16),
        ],
        scratch_shapes=[pltpu.VMEM((n, d1), jnp.float32)],
    )(g, f, W1)

    bm2 = 200
    p2 = n // bm2
    h2, h3, adj = pl.pallas_call(
        functools.partial(_rest_body, p=p2, block_m=bm2),
        grid=(3 * p2,),
        in_specs=[
            pl.BlockSpec(
                (bm2, n),
                lambda i: (jnp.where(i < p2, i,
                                     jnp.where(i < 2 * p2, i - p2, p2 - 1)), 0)),
            pl.BlockSpec((n, d1), lambda i: (0, 0)),
            pl.BlockSpec((d1, d2), lambda i: (0, 0)),
            pl.BlockSpec((d2, d3), lambda i: (0, 0)),
        ],
        out_specs=[
            pl.BlockSpec((bm2, d2), lambda i: (jnp.minimum(i, p2 - 1), 0)),
            pl.BlockSpec(
                (bm2, d3),
                lambda i: (jnp.where(i < p2, 0,
                                     jnp.minimum(i - p2, p2 - 1)), 0)),
            pl.BlockSpec(
                (bm2, n),
                lambda i: (jnp.where(i < 2 * p2, 0, i - 2 * p2), 0)),
        ],
        out_shape=[
            jax.ShapeDtypeStruct((n, d2), jnp.float32),
            jax.ShapeDtypeStruct((n, d3), jnp.float32),
            jax.ShapeDtypeStruct((n, n), jnp.float32),
        ],
        scratch_shapes=[
            pltpu.VMEM((n, 2 * d2), jnp.bfloat16),
            pltpu.VMEM((n, 2 * d3), jnp.bfloat16),
            pltpu.VMEM((n, d2), jnp.float32),
            pltpu.VMEM((n, d3), jnp.float32),
        ],
    )(gb, h1, W2, W3)

    return (h1, h3, adj, h2, h3)


# speedup vs baseline: 1.0285x; 1.0285x over previous
import jax
import jax.numpy as jnp
from jax.experimental import pallas as pl
from jax.experimental.pallas import tpu as pltpu


def _l1_body(g_ref, f_ref, w_ref, h1_ref, gb_ref, s_ref):
    @pl.when(pl.program_id(0) == 0)
    def _():
        s_ref[...] = jnp.dot(f_ref[...], w_ref[...],
                             preferred_element_type=jnp.float32)

    gblk = g_ref[...]
    m = jnp.dot(gblk, s_ref[...], preferred_element_type=jnp.float32)
    h1_ref[...] = jnp.tanh(m)
    gb_ref[...] = gblk.astype(jnp.bfloat16)


def _mid_body(gb_ref, x_ref, w_ref, out_ref, s_ref):
    @pl.when(pl.program_id(0) == 0)
    def _():
        s = jnp.dot(x_ref[...], w_ref[...],
                    preferred_element_type=jnp.float32)
        hi = s.astype(jnp.bfloat16)
        lo = (s - hi.astype(jnp.float32)).astype(jnp.bfloat16)
        s_ref[...] = jnp.concatenate([hi, lo], axis=1)

    acc = jnp.dot(gb_ref[...], s_ref[...], preferred_element_type=jnp.float32)
    ko = out_ref.shape[1]
    out_ref[...] = acc[:, :ko] + acc[:, ko:]


def _dec_body(a_ref, b_ref, adj_ref):
    adj_ref[...] = jax.lax.dot_general(
        a_ref[...], b_ref[...], (((1,), (1,)), ((), ())),
        preferred_element_type=jnp.float32)


def _mid_layer(gb, x, w, block_m):
    n = gb.shape[0]
    k = x.shape[1]
    ko = w.shape[1]
    return pl.pallas_call(
        _mid_body,
        grid=(n // block_m,),
        in_specs=[
            pl.BlockSpec((block_m, n), lambda i: (i, 0)),
            pl.BlockSpec((n, k), lambda i: (0, 0)),
            pl.BlockSpec((k, ko), lambda i: (0, 0)),
        ],
        out_specs=pl.BlockSpec((block_m, ko), lambda i: (i, 0)),
        out_shape=jax.ShapeDtypeStruct((n, ko), jnp.float32),
        scratch_shapes=[pltpu.VMEM((n, 2 * ko), jnp.bfloat16)],
    )(gb, x, w)


def kernel(g, f, W1, W2, W3):
    n = g.shape[0]
    d0 = f.shape[1]
    d1 = W1.shape[1]

    block_m = 400
    h1, gb = pl.pallas_call(
        _l1_body,
        grid=(n // block_m,),
        in_specs=[
            pl.BlockSpec((block_m, n), lambda i: (i, 0)),
            pl.BlockSpec((n, d0), lambda i: (0, 0)),
            pl.BlockSpec((d0, d1), lambda i: (0, 0)),
        ],
        out_specs=[
            pl.BlockSpec((block_m, d1), lambda i: (i, 0)),
            pl.BlockSpec((block_m, n), lambda i: (i, 0)),
        ],
        out_shape=[
            jax.ShapeDtypeStruct((n, d1), jnp.float32),
            jax.ShapeDtypeStruct((n, n), jnp.bfloat16),
        ],
        scratch_shapes=[pltpu.VMEM((n, d1), jnp.float32)],
    )(g, f, W1)

    h2 = _mid_layer(gb, h1, W2, block_m)
    h3 = _mid_layer(gb, h2, W3, block_m)

    block_r, block_c = 2000, 2048
    adj = pl.pallas_call(
        _dec_body,
        grid=(n // block_r, pl.cdiv(n, block_c)),
        in_specs=[
            pl.BlockSpec((block_r, W3.shape[1]), lambda i, j: (i, 0)),
            pl.BlockSpec((block_c, W3.shape[1]), lambda i, j: (j, 0)),
        ],
        out_specs=pl.BlockSpec((block_r, block_c), lambda i, j: (i, j)),
        out_shape=jax.ShapeDtypeStruct((n, n), jnp.float32),
    )(h3, h3)

    return (h1, h3, adj, h2, h3)


# speedup vs baseline: 1.0518x; 1.0227x over previous
import functools

import jax
import jax.numpy as jnp
from jax.experimental import pallas as pl
from jax.experimental.pallas import tpu as pltpu


def _hilo(s):
    hi = s.astype(jnp.bfloat16)
    lo = (s - hi.astype(jnp.float32)).astype(jnp.bfloat16)
    return jnp.concatenate([hi, lo], axis=1)


def _dot_nt(a, b):
    return jax.lax.dot_general(a, b, (((1,), (1,)), ((), ())),
                               preferred_element_type=jnp.float32)


def _l1_body(g_ref, f_ref, w1t_ref, h1_ref, gb_ref, s_ref):
    @pl.when(pl.program_id(0) == 0)
    def _():
        s_ref[...] = _dot_nt(f_ref[...], w1t_ref[...])

    gblk = g_ref[...]
    m = jnp.dot(gblk, s_ref[...], preferred_element_type=jnp.float32)
    h1_ref[...] = jnp.tanh(m)
    gb_ref[...] = gblk.astype(jnp.bfloat16)


def _mid_body(gb_ref, x_ref, wt_ref, out_ref, xt_ref, s_ref):
    @pl.when(pl.program_id(0) == 0)
    def _():
        x = x_ref[...]
        s_ref[...] = _hilo(_dot_nt(x, wt_ref[...]))
        xt_ref[...] = x.T

    acc = jnp.dot(gb_ref[...], s_ref[...], preferred_element_type=jnp.float32)
    ko = out_ref.shape[1]
    out_ref[...] = acc[:, :ko] + acc[:, ko:]


def _dec_body(a_ref, b_ref, h3_ref, adj_ref, h3t_ref, *, first):
    @pl.when(jnp.logical_and(pl.program_id(0) == first,
                             pl.program_id(1) == 0))
    def _():
        h3t_ref[...] = h3_ref[...].T

    adj_ref[...] = _dot_nt(a_ref[...], b_ref[...])


def _mid_layer(gb, x, wt, block_m):
    n = gb.shape[0]
    ko = wt.shape[0]
    return pl.pallas_call(
        _mid_body,
        grid=(n // block_m,),
        in_specs=[
            pl.BlockSpec((block_m, n), lambda i: (i, 0)),
            pl.BlockSpec(x.shape, lambda i: (0, 0)),
            pl.BlockSpec(wt.shape, lambda i: (0, 0)),
        ],
        out_specs=[
            pl.BlockSpec((block_m, ko), lambda i: (i, 0)),
            pl.BlockSpec((x.shape[1], n), lambda i: (0, 0)),
        ],
        out_shape=[
            jax.ShapeDtypeStruct((n, ko), jnp.float32),
            jax.ShapeDtypeStruct((x.shape[1], n), jnp.float32),
        ],
        scratch_shapes=[pltpu.VMEM((n, 2 * ko), jnp.bfloat16)],
    )(gb, x, wt)


def kernel(g, f, W1, W2, W3):
    n = g.shape[0]
    d0 = f.shape[1]
    d1, d2, d3 = W1.shape[1], W2.shape[1], W3.shape[1]
    w1t, w2t, w3t = W1.T, W2.T, W3.T

    block_m = 400
    h1, gb = pl.pallas_call(
        _l1_body,
        grid=(n // block_m,),
        in_specs=[
            pl.BlockSpec((block_m, n), lambda i: (i, 0)),
            pl.BlockSpec((n, d0), lambda i: (0, 0)),
            pl.BlockSpec((d1, d0), lambda i: (0, 0)),
        ],
        out_specs=[
            pl.BlockSpec((block_m, d1), lambda i: (i, 0)),
            pl.BlockSpec((block_m, n), lambda i: (i, 0)),
        ],
        out_shape=[
            jax.ShapeDtypeStruct((n, d1), jnp.float32),
            jax.ShapeDtypeStruct((n, n), jnp.bfloat16),
        ],
        scratch_shapes=[pltpu.VMEM((n, d1), jnp.float32)],
    )(g, f, w1t)

    h2, h1t = _mid_layer(gb, h1, w2t, block_m)
    h3, h2t = _mid_layer(gb, h2, w3t, block_m)

    block_r, block_c = 2000, 2048
    adj, h3t = pl.pallas_call(
        functools.partial(_dec_body, first=0),
        grid=(n // block_r, pl.cdiv(n, block_c)),
        in_specs=[
            pl.BlockSpec((block_r, d3), lambda i, j: (i, 0)),
            pl.BlockSpec((block_c, d3), lambda i, j: (j, 0)),
            pl.BlockSpec((n, d3), lambda i, j: (0, 0)),
        ],
        out_specs=[
            pl.BlockSpec((block_r, block_c), lambda i, j: (i, j)),
            pl.BlockSpec((d3, n), lambda i, j: (0, 0)),
        ],
        out_shape=[
            jax.ShapeDtypeStruct((n, n), jnp.float32),
            jax.ShapeDtypeStruct((d3, n), jnp.float32),
        ],
    )(h3, h3, h3)

    h3o = h3t.T
    return (h1t.T, h3o, adj, h2t.T, h3o)


# speedup vs baseline: 1.0846x; 1.0312x over previous
import functools

import jax
import jax.numpy as jnp
from jax.experimental import pallas as pl
from jax.experimental.pallas import tpu as pltpu


def _hilo(s):
    hi = s.astype(jnp.bfloat16)
    lo = (s - hi.astype(jnp.float32)).astype(jnp.bfloat16)
    return jnp.concatenate([hi, lo], axis=1)


def _dot_nt(a, b):
    return jax.lax.dot_general(a, b, (((1,), (1,)), ((), ())),
                               preferred_element_type=jnp.float32)


def _l1_body(g_ref, f_ref, w1t_ref, h1_ref, gb_ref, s_ref):
    @pl.when(pl.program_id(0) == 0)
    def _():
        s_ref[...] = _dot_nt(f_ref[...], w1t_ref[...])

    gblk = g_ref[...]
    m = jnp.dot(gblk, s_ref[...], preferred_element_type=jnp.float32)
    h1_ref[...] = jnp.tanh(m)
    gb_ref[...] = gblk.astype(jnp.bfloat16)


def _mid_body(gb_ref, x_ref, wt_ref, out_ref, xt_ref, s_ref):
    @pl.when(pl.program_id(0) == 0)
    def _():
        x = x_ref[...]
        s_ref[...] = _hilo(_dot_nt(x, wt_ref[...]))
        xt_ref[...] = x.T

    acc = jnp.dot(gb_ref[...], s_ref[...], preferred_element_type=jnp.float32)
    ko = out_ref.shape[1]
    out_ref[...] = acc[:, :ko] + acc[:, ko:]


def _dec_body(a_ref, b_ref, h3_ref, adj_ref, h3t_ref, *, first):
    @pl.when(jnp.logical_and(pl.program_id(0) == first,
                             pl.program_id(1) == 0))
    def _():
        h3t_ref[...] = h3_ref[...].T

    adj_ref[...] = _dot_nt(a_ref[...], b_ref[...])


def _mid_layer(gb, x, wt, block_m):
    n = gb.shape[0]
    ko = wt.shape[0]
    return pl.pallas_call(
        _mid_body,
        grid=(n // block_m,),
        in_specs=[
            pl.BlockSpec((block_m, n), lambda i: (i, 0)),
            pl.BlockSpec(x.shape, lambda i: (0, 0)),
            pl.BlockSpec(wt.shape, lambda i: (0, 0)),
        ],
        out_specs=[
            pl.BlockSpec((block_m, ko), lambda i: (i, 0)),
            pl.BlockSpec((x.shape[1], n), lambda i: (0, 0)),
        ],
        out_shape=[
            jax.ShapeDtypeStruct((n, ko), jnp.float32),
            jax.ShapeDtypeStruct((x.shape[1], n), jnp.float32),
        ],
        scratch_shapes=[pltpu.VMEM((n, 2 * ko), jnp.bfloat16)],
    )(gb, x, wt)


def kernel(g, f, W1, W2, W3):
    n = g.shape[0]
    d0 = f.shape[1]
    d1, d2, d3 = W1.shape[1], W2.shape[1], W3.shape[1]
    w1t, w2t, w3t = W1.T, W2.T, W3.T

    block_m = 400
    h1, gb = pl.pallas_call(
        _l1_body,
        grid=(n // block_m,),
        in_specs=[
            pl.BlockSpec((block_m, n), lambda i: (i, 0)),
            pl.BlockSpec((n, d0), lambda i: (0, 0)),
            pl.BlockSpec((d1, d0), lambda i: (0, 0)),
        ],
        out_specs=[
            pl.BlockSpec((block_m, d1), lambda i: (i, 0)),
            pl.BlockSpec((block_m, n), lambda i: (i, 0)),
        ],
        out_shape=[
            jax.ShapeDtypeStruct((n, d1), jnp.float32),
            jax.ShapeDtypeStruct((n, n), jnp.bfloat16),
        ],
        scratch_shapes=[pltpu.VMEM((n, d1), jnp.float32)],
    )(g, f, w1t)

    h2, h1t = _mid_layer(gb, h1, w2t, 1000)
    h3, h2t = _mid_layer(gb, h2, w3t, 1000)

    block_r, block_c = 2000, 2048
    adj, h3t = pl.pallas_call(
        functools.partial(_dec_body, first=0),
        grid=(n // block_r, pl.cdiv(n, block_c)),
        in_specs=[
            pl.BlockSpec((block_r, d3), lambda i, j: (i, 0)),
            pl.BlockSpec((block_c, d3), lambda i, j: (j, 0)),
            pl.BlockSpec((n, d3), lambda i, j: (0, 0)),
        ],
        out_specs=[
            pl.BlockSpec((block_r, block_c), lambda i, j: (i, j)),
            pl.BlockSpec((d3, n), lambda i, j: (0, 0)),
        ],
        out_shape=[
            jax.ShapeDtypeStruct((n, n), jnp.float32),
            jax.ShapeDtypeStruct((d3, n), jnp.float32),
        ],
    )(h3, h3, h3)

    h3o = h3t.T
    return (h1t.T, h3o, adj, h2t.T, h3o)


# speedup vs baseline: 1.1082x; 1.0218x over previous
import functools

import jax
import jax.numpy as jnp
from jax.experimental import pallas as pl
from jax.experimental.pallas import tpu as pltpu


def _hilo(s):
    hi = s.astype(jnp.bfloat16)
    lo = (s - hi.astype(jnp.float32)).astype(jnp.bfloat16)
    return jnp.concatenate([hi, lo], axis=1)


def _dot_nt(a, b):
    return jax.lax.dot_general(a, b, (((1,), (1,)), ((), ())),
                               preferred_element_type=jnp.float32)


def _l1_body(g_ref, f_ref, w1t_ref, h1_ref, gb_ref, s_ref):
    @pl.when(pl.program_id(0) == 0)
    def _():
        s_ref[...] = _dot_nt(f_ref[...], w1t_ref[...])

    gblk = g_ref[...]
    m = jnp.dot(gblk, s_ref[...], preferred_element_type=jnp.float32)
    h1_ref[...] = jnp.tanh(m)
    gb_ref[...] = gblk.astype(jnp.bfloat16)


def _mid_body(gb_ref, x_ref, wt_ref, out_ref, xt_ref, s_ref):
    @pl.when(pl.program_id(0) == 0)
    def _():
        x = x_ref[...]
        s_ref[...] = _hilo(_dot_nt(x, wt_ref[...]))
        xt_ref[...] = x.T

    acc = jnp.dot(gb_ref[...], s_ref[...], preferred_element_type=jnp.float32)
    ko = out_ref.shape[1]
    out_ref[...] = acc[:, :ko] + acc[:, ko:]


def _dec_body(a_ref, b_ref, adj_ref, h3t_ref):
    @pl.when(pl.program_id(0) == 0)
    def _():
        h3t_ref[...] = b_ref[...].T

    adj_ref[...] = _dot_nt(a_ref[...], b_ref[...])


def _mid_layer(gb, x, wt, block_m):
    n = gb.shape[0]
    ko = wt.shape[0]
    return pl.pallas_call(
        _mid_body,
        grid=(n // block_m,),
        in_specs=[
            pl.BlockSpec((block_m, n), lambda i: (i, 0)),
            pl.BlockSpec(x.shape, lambda i: (0, 0)),
            pl.BlockSpec(wt.shape, lambda i: (0, 0)),
        ],
        out_specs=[
            pl.BlockSpec((block_m, ko), lambda i: (i, 0)),
            pl.BlockSpec((x.shape[1], n), lambda i: (0, 0)),
        ],
        out_shape=[
            jax.ShapeDtypeStruct((n, ko), jnp.float32),
            jax.ShapeDtypeStruct((x.shape[1], n), jnp.float32),
        ],
        scratch_shapes=[pltpu.VMEM((n, 2 * ko), jnp.bfloat16)],
    )(gb, x, wt)


def kernel(g, f, W1, W2, W3):
    n = g.shape[0]
    d0 = f.shape[1]
    d1, d2, d3 = W1.shape[1], W2.shape[1], W3.shape[1]
    w1t, w2t, w3t = W1.T, W2.T, W3.T

    block_m = 400
    h1, gb = pl.pallas_call(
        _l1_body,
        grid=(n // block_m,),
        in_specs=[
            pl.BlockSpec((block_m, n), lambda i: (i, 0)),
            pl.BlockSpec((n, d0), lambda i: (0, 0)),
            pl.BlockSpec((d1, d0), lambda i: (0, 0)),
        ],
        out_specs=[
            pl.BlockSpec((block_m, d1), lambda i: (i, 0)),
            pl.BlockSpec((block_m, n), lambda i: (i, 0)),
        ],
        out_shape=[
            jax.ShapeDtypeStruct((n, d1), jnp.float32),
            jax.ShapeDtypeStruct((n, n), jnp.bfloat16),
        ],
        scratch_shapes=[pltpu.VMEM((n, d1), jnp.float32)],
    )(g, f, w1t)

    h2, h1t = _mid_layer(gb, h1, w2t, 1000)
    h3, h2t = _mid_layer(gb, h2, w3t, 1000)

    block_r = 400
    adj, h3t = pl.pallas_call(
        _dec_body,
        grid=(n // block_r,),
        in_specs=[
            pl.BlockSpec((block_r, d3), lambda i: (i, 0)),
            pl.BlockSpec((n, d3), lambda i: (0, 0)),
        ],
        out_specs=[
            pl.BlockSpec((block_r, n), lambda i: (i, 0)),
            pl.BlockSpec((d3, n), lambda i: (0, 0)),
        ],
        out_shape=[
            jax.ShapeDtypeStruct((n, n), jnp.float32),
            jax.ShapeDtypeStruct((d3, n), jnp.float32),
        ],
    )(h3, h3)

    h3o = h3t.T
    return (h1t.T, h3o, adj, h2t.T, h3o)


# speedup vs baseline: 1.1222x; 1.0126x over previous
import functools

import jax
import jax.numpy as jnp
from jax.experimental import pallas as pl
from jax.experimental.pallas import tpu as pltpu


def _hilo(s):
    hi = s.astype(jnp.bfloat16)
    lo = (s - hi.astype(jnp.float32)).astype(jnp.bfloat16)
    return jnp.concatenate([hi, lo], axis=1)


def _dot_nt(a, b):
    return jax.lax.dot_general(a, b, (((1,), (1,)), ((), ())),
                               preferred_element_type=jnp.float32)


def _l1_body(g_ref, f_ref, w1t_ref, h1_ref, gb_ref, s_ref):
    @pl.when(pl.program_id(0) == 0)
    def _():
        s_ref[...] = _dot_nt(f_ref[...], w1t_ref[...])

    gblk = g_ref[...]
    m = jnp.dot(gblk, s_ref[...], preferred_element_type=jnp.float32)
    h1_ref[...] = jnp.tanh(m)
    gb_ref[...] = gblk.astype(jnp.bfloat16)


def _mids_body(gb_ref, h1_ref, w2t_ref, w3t_ref,
               h2_ref, h3_ref,
               s2_ref, s3_ref, h2s_ref, *, p, block_m):
    i = pl.program_id(0)

    @pl.when(i == 0)
    def _():
        s2_ref[...] = _hilo(_dot_nt(h1_ref[...], w2t_ref[...]))

    @pl.when(i < p)
    def _():
        acc = jnp.dot(gb_ref[...], s2_ref[...],
                      preferred_element_type=jnp.float32)
        ko = h2_ref.shape[1]
        m2 = acc[:, :ko] + acc[:, ko:]
        h2_ref[...] = m2
        h2s_ref[pl.ds(i * block_m, block_m), :] = m2

    @pl.when(i == p)
    def _():
        s3_ref[...] = _hilo(_dot_nt(h2s_ref[...], w3t_ref[...]))

    @pl.when(i >= p)
    def _():
        acc = jnp.dot(gb_ref[...], s3_ref[...],
                      preferred_element_type=jnp.float32)
        ko = h3_ref.shape[1]
        h3_ref[...] = acc[:, :ko] + acc[:, ko:]


def _dec_body(a_ref, b_ref, h1_ref, h2_ref, adj_ref,
              h1t_ref, h2t_ref, h3t_ref):
    @pl.when(pl.program_id(0) == 0)
    def _():
        h1t_ref[...] = h1_ref[...].T
        h2t_ref[...] = h2_ref[...].T
        h3t_ref[...] = b_ref[...].T

    adj_ref[...] = _dot_nt(a_ref[...], b_ref[...])


def _mid_layers(gb, h1, w2t, w3t, block_m):
    n = gb.shape[0]
    d1, d2, d3 = h1.shape[1], w2t.shape[0], w3t.shape[0]
    p = n // block_m
    return pl.pallas_call(
        functools.partial(_mids_body, p=p, block_m=block_m),
        grid=(2 * p,),
        in_specs=[
            pl.BlockSpec((block_m, n),
                         lambda i: (jnp.where(i < p, i, i - p), 0)),
            pl.BlockSpec(h1.shape, lambda i: (0, 0)),
            pl.BlockSpec(w2t.shape, lambda i: (0, 0)),
            pl.BlockSpec(w3t.shape, lambda i: (0, 0)),
        ],
        out_specs=[
            pl.BlockSpec((block_m, d2), lambda i: (jnp.minimum(i, p - 1), 0)),
            pl.BlockSpec(
                (block_m, d3),
                lambda i: (jnp.where(i < p, 0, i - p), 0)),
        ],
        out_shape=[
            jax.ShapeDtypeStruct((n, d2), jnp.float32),
            jax.ShapeDtypeStruct((n, d3), jnp.float32),
        ],
        scratch_shapes=[
            pltpu.VMEM((n, 2 * d2), jnp.bfloat16),
            pltpu.VMEM((n, 2 * d3), jnp.bfloat16),
            pltpu.VMEM((n, d2), jnp.float32),
        ],
    )(gb, h1, w2t, w3t)


def kernel(g, f, W1, W2, W3):
    n = g.shape[0]
    d0 = f.shape[1]
    d1, d2, d3 = W1.shape[1], W2.shape[1], W3.shape[1]
    w1t, w2t, w3t = W1.T, W2.T, W3.T

    block_m = 400
    h1, gb = pl.pallas_call(
        _l1_body,
        grid=(n // block_m,),
        in_specs=[
            pl.BlockSpec((block_m, n), lambda i: (i, 0)),
            pl.BlockSpec((n, d0), lambda i: (0, 0)),
            pl.BlockSpec((d1, d0), lambda i: (0, 0)),
        ],
        out_specs=[
            pl.BlockSpec((block_m, d1), lambda i: (i, 0)),
            pl.BlockSpec((block_m, n), lambda i: (i, 0)),
        ],
        out_shape=[
            jax.ShapeDtypeStruct((n, d1), jnp.float32),
            jax.ShapeDtypeStruct((n, n), jnp.bfloat16),
        ],
        scratch_shapes=[pltpu.VMEM((n, d1), jnp.float32)],
    )(g, f, w1t)

    h2, h3 = _mid_layers(gb, h1, w2t, w3t, 1000)

    block_r = 400
    adj, h1t, h2t, h3t = pl.pallas_call(
        _dec_body,
        grid=(n // block_r,),
        in_specs=[
            pl.BlockSpec((block_r, d3), lambda i: (i, 0)),
            pl.BlockSpec((n, d3), lambda i: (0, 0)),
            pl.BlockSpec((n, d1), lambda i: (0, 0)),
            pl.BlockSpec((n, d2), lambda i: (0, 0)),
        ],
        out_specs=[
            pl.BlockSpec((block_r, n), lambda i: (i, 0)),
            pl.BlockSpec((d1, n), lambda i: (0, 0)),
            pl.BlockSpec((d2, n), lambda i: (0, 0)),
            pl.BlockSpec((d3, n), lambda i: (0, 0)),
        ],
        out_shape=[
            jax.ShapeDtypeStruct((n, n), jnp.float32),
            jax.ShapeDtypeStruct((d1, n), jnp.float32),
            jax.ShapeDtypeStruct((d2, n), jnp.float32),
            jax.ShapeDtypeStruct((d3, n), jnp.float32),
        ],
    )(h3, h3, h1, h2)

    h3o = h3t.T
    return (h1t.T, h3o, adj, h2t.T, h3o)


# speedup vs baseline: 1.1284x; 1.0055x over previous
import functools

import jax
import jax.numpy as jnp
from jax.experimental import pallas as pl
from jax.experimental.pallas import tpu as pltpu


def _hilo(s):
    hi = s.astype(jnp.bfloat16)
    lo = (s - hi.astype(jnp.float32)).astype(jnp.bfloat16)
    return jnp.concatenate([hi, lo], axis=1)


def _dot_nt(a, b):
    return jax.lax.dot_general(a, b, (((1,), (1,)), ((), ())),
                               preferred_element_type=jnp.float32)


def _l1_body(g_ref, f_ref, w1t_ref, h1_ref, gb_ref, s_ref):
    @pl.when(pl.program_id(0) == 0)
    def _():
        s_ref[...] = _dot_nt(f_ref[...], w1t_ref[...])

    gblk = g_ref[...]
    m = jnp.dot(gblk, s_ref[...], preferred_element_type=jnp.float32)
    h1_ref[...] = jnp.tanh(m)
    gb_ref[...] = gblk.astype(jnp.bfloat16)


def _mids_body(gb_ref, h1_ref, w2t_ref, w3t_ref,
               h2_ref, h3_ref,
               s2_ref, s3_ref, h2s_ref, *, p, block_m):
    i = pl.program_id(0)

    @pl.when(i == 0)
    def _():
        s2_ref[...] = _hilo(_dot_nt(h1_ref[...], w2t_ref[...]))

    @pl.when(i < p)
    def _():
        acc = jnp.dot(gb_ref[...], s2_ref[...],
                      preferred_element_type=jnp.float32)
        ko = h2_ref.shape[1]
        m2 = acc[:, :ko] + acc[:, ko:]
        h2_ref[...] = m2
        h2s_ref[pl.ds(i * block_m, block_m), :] = m2

    @pl.when(i == p)
    def _():
        s3_ref[...] = _hilo(_dot_nt(h2s_ref[...], w3t_ref[...]))

    @pl.when(i >= p)
    def _():
        acc = jnp.dot(gb_ref[...], s3_ref[...],
                      preferred_element_type=jnp.float32)
        ko = h3_ref.shape[1]
        h3_ref[...] = acc[:, :ko] + acc[:, ko:]


def _dec_body(a_ref, b_ref, h1_ref, h2_ref, adj_ref,
              h1t_ref, h2t_ref, h3t_ref, m3t_ref):
    i = pl.program_id(0)

    @pl.when(i == 0)
    def _():
        h3t_ref[...] = b_ref[...].T

    @pl.when(i == 1)
    def _():
        m3t_ref[...] = b_ref[...].T

    @pl.when(i == 2)
    def _():
        h1t_ref[...] = h1_ref[...].T

    @pl.when(i == 3)
    def _():
        h2t_ref[...] = h2_ref[...].T

    adj_ref[...] = _dot_nt(a_ref[...], b_ref[...])


def _mid_layers(gb, h1, w2t, w3t, block_m):
    n = gb.shape[0]
    d1, d2, d3 = h1.shape[1], w2t.shape[0], w3t.shape[0]
    p = n // block_m
    return pl.pallas_call(
        functools.partial(_mids_body, p=p, block_m=block_m),
        grid=(2 * p,),
        in_specs=[
            pl.BlockSpec((block_m, n),
                         lambda i: (jnp.where(i < p, i, i - p), 0)),
            pl.BlockSpec(h1.shape, lambda i: (0, 0)),
            pl.BlockSpec(w2t.shape, lambda i: (0, 0)),
            pl.BlockSpec(w3t.shape, lambda i: (0, 0)),
        ],
        out_specs=[
            pl.BlockSpec((block_m, d2), lambda i: (jnp.minimum(i, p - 1), 0)),
            pl.BlockSpec(
                (block_m, d3),
                lambda i: (jnp.where(i < p, 0, i - p), 0)),
        ],
        out_shape=[
            jax.ShapeDtypeStruct((n, d2), jnp.float32),
            jax.ShapeDtypeStruct((n, d3), jnp.float32),
        ],
        scratch_shapes=[
            pltpu.VMEM((n, 2 * d2), jnp.bfloat16),
            pltpu.VMEM((n, 2 * d3), jnp.bfloat16),
            pltpu.VMEM((n, d2), jnp.float32),
        ],
    )(gb, h1, w2t, w3t)


def kernel(g, f, W1, W2, W3):
    n = g.shape[0]
    d0 = f.shape[1]
    d1, d2, d3 = W1.shape[1], W2.shape[1], W3.shape[1]
    w1t, w2t, w3t = W1.T, W2.T, W3.T

    block_m = 400
    h1, gb = pl.pallas_call(
        _l1_body,
        grid=(n // block_m,),
        in_specs=[
            pl.BlockSpec((block_m, n), lambda i: (i, 0)),
            pl.BlockSpec((n, d0), lambda i: (0, 0)),
            pl.BlockSpec((d1, d0), lambda i: (0, 0)),
        ],
        out_specs=[
            pl.BlockSpec((block_m, d1), lambda i: (i, 0)),
            pl.BlockSpec((block_m, n), lambda i: (i, 0)),
        ],
        out_shape=[
            jax.ShapeDtypeStruct((n, d1), jnp.float32),
            jax.ShapeDtypeStruct((n, n), jnp.bfloat16),
        ],
        scratch_shapes=[pltpu.VMEM((n, d1), jnp.float32)],
    )(g, f, w1t)

    h2, h3 = _mid_layers(gb, h1, w2t, w3t, 1000)

    block_r = 400
    adj, h1t, h2t, h3t, m3t = pl.pallas_call(
        _dec_body,
        grid=(n // block_r,),
        in_specs=[
            pl.BlockSpec((block_r, d3), lambda i: (i, 0)),
            pl.BlockSpec((n, d3), lambda i: (0, 0)),
            pl.BlockSpec((n, d1), lambda i: (0, 0)),
            pl.BlockSpec((n, d2), lambda i: (0, 0)),
        ],
        out_specs=[
            pl.BlockSpec((block_r, n), lambda i: (i, 0)),
            pl.BlockSpec((d1, n), lambda i: (0, 0)),
            pl.BlockSpec((d2, n), lambda i: (0, 0)),
            pl.BlockSpec((d3, n), lambda i: (0, 0)),
            pl.BlockSpec((d3, n), lambda i: (0, 0)),
        ],
        out_shape=[
            jax.ShapeDtypeStruct((n, n), jnp.float32),
            jax.ShapeDtypeStruct((d1, n), jnp.float32),
            jax.ShapeDtypeStruct((d2, n), jnp.float32),
            jax.ShapeDtypeStruct((d3, n), jnp.float32),
            jax.ShapeDtypeStruct((d3, n), jnp.float32),
        ],
    )(h3, h3, h1, h2)

    return (h1t.T, h3t.T, adj, h2t.T, m3t.T)


# speedup vs baseline: 1.1298x; 1.0013x over previous
import functools

import jax
import jax.numpy as jnp
from jax.experimental import pallas as pl
from jax.experimental.pallas import tpu as pltpu


def _hilo(s):
    hi = s.astype(jnp.bfloat16)
    lo = (s - hi.astype(jnp.float32)).astype(jnp.bfloat16)
    return jnp.concatenate([hi, lo], axis=1)


def _dot_nt(a, b):
    return jax.lax.dot_general(a, b, (((1,), (1,)), ((), ())),
                               preferred_element_type=jnp.float32)


def _l1_body(g_ref, f_ref, w1t_ref, h1_ref, gb_ref, s_ref):
    @pl.when(pl.program_id(0) == 0)
    def _():
        s_ref[...] = _dot_nt(f_ref[...], w1t_ref[...])

    gblk = g_ref[...]
    m = jnp.dot(gblk, s_ref[...], preferred_element_type=jnp.float32)
    h1_ref[...] = jnp.tanh(m)
    gb_ref[...] = gblk.astype(jnp.bfloat16)


def _hilo_t(st):
    hi = st.astype(jnp.bfloat16)
    lo = (st - hi.astype(jnp.float32)).astype(jnp.bfloat16)
    return jnp.concatenate([hi, lo], axis=0)


def _mids_body(gb_ref, h1_ref, w2t_ref, w3t_ref,
               h2_ref, h3_ref,
               s2_ref, s3_ref, h2s_ref, *, p, block_m):
    i = pl.program_id(0)

    @pl.when(i == 0)
    def _():
        s2_ref[...] = _hilo_t(_dot_nt(w2t_ref[...], h1_ref[...]))

    @pl.when(i < p)
    def _():
        acc = _dot_nt(gb_ref[...], s2_ref[...])
        ko = h2_ref.shape[1]
        m2 = acc[:, :ko] + acc[:, ko:]
        h2_ref[...] = m2
        h2s_ref[pl.ds(i * block_m, block_m), :] = m2

    @pl.when(i == p)
    def _():
        s3_ref[...] = _hilo_t(_dot_nt(w3t_ref[...], h2s_ref[...]))

    @pl.when(i >= p)
    def _():
        acc = _dot_nt(gb_ref[...], s3_ref[...])
        ko = h3_ref.shape[1]
        h3_ref[...] = acc[:, :ko] + acc[:, ko:]


def _dec_body(a_ref, b_ref, h1_ref, h2_ref, adj_ref,
              h1t_ref, h2t_ref, h3t_ref, m3t_ref):
    i = pl.program_id(0)

    @pl.when(i == 0)
    def _():
        h3t_ref[...] = b_ref[...].T

    @pl.when(i == 1)
    def _():
        m3t_ref[...] = b_ref[...].T

    @pl.when(i == 2)
    def _():
        h1t_ref[...] = h1_ref[...].T

    @pl.when(i == 3)
    def _():
        h2t_ref[...] = h2_ref[...].T

    adj_ref[...] = _dot_nt(a_ref[...], b_ref[...])


def _mid_layers(gb, h1, w2t, w3t, block_m):
    n = gb.shape[0]
    d1, d2, d3 = h1.shape[1], w2t.shape[0], w3t.shape[0]
    p = n // block_m
    return pl.pallas_call(
        functools.partial(_mids_body, p=p, block_m=block_m),
        grid=(2 * p,),
        in_specs=[
            pl.BlockSpec((block_m, n),
                         lambda i: (jnp.where(i < p, i, i - p), 0)),
            pl.BlockSpec(h1.shape, lambda i: (0, 0)),
            pl.BlockSpec(w2t.shape, lambda i: (0, 0)),
            pl.BlockSpec(w3t.shape, lambda i: (0, 0)),
        ],
        out_specs=[
            pl.BlockSpec((block_m, d2), lambda i: (jnp.minimum(i, p - 1), 0)),
            pl.BlockSpec(
                (block_m, d3),
                lambda i: (jnp.where(i < p, 0, i - p), 0)),
        ],
        out_shape=[
            jax.ShapeDtypeStruct((n, d2), jnp.float32),
            jax.ShapeDtypeStruct((n, d3), jnp.float32),
        ],
        scratch_shapes=[
            pltpu.VMEM((2 * d2, n), jnp.bfloat16),
            pltpu.VMEM((2 * d3, n), jnp.bfloat16),
            pltpu.VMEM((n, d2), jnp.float32),
        ],
    )(gb, h1, w2t, w3t)


def kernel(g, f, W1, W2, W3):
    n = g.shape[0]
    d0 = f.shape[1]
    d1, d2, d3 = W1.shape[1], W2.shape[1], W3.shape[1]
    w1t, w2t, w3t = W1.T, W2.T, W3.T

    block_m = 400
    h1, gb = pl.pallas_call(
        _l1_body,
        grid=(n // block_m,),
        in_specs=[
            pl.BlockSpec((block_m, n), lambda i: (i, 0)),
            pl.BlockSpec((n, d0), lambda i: (0, 0)),
            pl.BlockSpec((d1, d0), lambda i: (0, 0)),
        ],
        out_specs=[
            pl.BlockSpec((block_m, d1), lambda i: (i, 0)),
            pl.BlockSpec((block_m, n), lambda i: (i, 0)),
        ],
        out_shape=[
            jax.ShapeDtypeStruct((n, d1), jnp.float32),
            jax.ShapeDtypeStruct((n, n), jnp.bfloat16),
        ],
        scratch_shapes=[pltpu.VMEM((n, d1), jnp.float32)],
    )(g, f, w1t)

    h2, h3 = _mid_layers(gb, h1, w2t, w3t, 1000)

    block_r = 400
    adj, h1t, h2t, h3t, m3t = pl.pallas_call(
        _dec_body,
        grid=(n // block_r,),
        in_specs=[
            pl.BlockSpec((block_r, d3), lambda i: (i, 0)),
            pl.BlockSpec((n, d3), lambda i: (0, 0)),
            pl.BlockSpec((n, d1), lambda i: (0, 0)),
            pl.BlockSpec((n, d2), lambda i: (0, 0)),
        ],
        out_specs=[
            pl.BlockSpec((block_r, n), lambda i: (i, 0)),
            pl.BlockSpec((d1, n), lambda i: (0, 0)),
            pl.BlockSpec((d2, n), lambda i: (0, 0)),
            pl.BlockSpec((d3, n), lambda i: (0, 0)),
            pl.BlockSpec((d3, n), lambda i: (0, 0)),
        ],
        out_shape=[
            jax.ShapeDtypeStruct((n, n), jnp.float32),
            jax.ShapeDtypeStruct((d1, n), jnp.float32),
            jax.ShapeDtypeStruct((d2, n), jnp.float32),
            jax.ShapeDtypeStruct((d3, n), jnp.float32),
            jax.ShapeDtypeStruct((d3, n), jnp.float32),
        ],
    )(h3, h3, h1, h2)

    return (h1t.T, h3t.T, adj, h2t.T, m3t.T)
